# baseline (speedup 1.0000x reference)
import functools

import jax
import jax.numpy as jnp
from jax import lax
from jax.experimental import pallas as pl
from jax.experimental.pallas import tpu as pltpu
from jax.experimental.pallas import tpu_sc as plsc

N, D, G, C = 10000, 128, 16, 7
NW = 32
NS = 16
CHUNK = 320
TAIL = N - 31 * CHUNK
GRP = 64
NG = CHUNK // GRP
TG0, TG1 = 64, TAIL - 64

_MESH = plsc.VectorSubcoreMesh(core_axis_name="c", subcore_axis_name="s")


@functools.partial(
    pl.kernel,
    mesh=_MESH,
    out_type=jax.ShapeDtypeStruct((2 * G, D), jnp.float32),
    scratch_types=(
        [pltpu.VMEM((CHUNK, D), jnp.float32)]
        + [pltpu.VMEM((GRP,), jnp.int32) for _ in range(NG)]
        + [pltpu.VMEM((TG1,), jnp.int32)]
        + [pltpu.VMEM((G, D), jnp.float32)]
        + [pltpu.VMEM((D,), jnp.float32)]
        + [pltpu.VMEM_SHARED((NS * G, D), jnp.float32)]
        + [pltpu.SemaphoreType.DMA for _ in range(NG)]
        + [pltpu.SemaphoreType.DMA,
           pltpu.SemaphoreType.DMA,
           pltpu.SemaphoreType.DMA]
    ),
)
def _sc_segment_sums(x_hbm, ids_hbm, sums_out,
                     x_v, i0, i1, i2, i3, i4, it_v, zs_v, row_v, acc_sh,
                     s0, s1, s2, s3, s4, si, ss, sg):
    idx_bufs = [i0, i1, i2, i3, i4]
    x_sems = [s0, s1, s2, s3, s4]
    cid = lax.axis_index("c")
    sid = lax.axis_index("s")
    wid = sid * 2 + cid
    base = wid * CHUNK
    is_tail = wid == NW - 1
    slab = sid * G

    @pl.when(~is_tail)
    def _stage():
        for g in range(NG):
            pltpu.async_copy(ids_hbm.at[pl.ds(base + GRP * g, GRP)],
                             idx_bufs[g], si)
        for g in range(NG):
            pltpu.async_copy(x_hbm.at[pl.ds(base + GRP * g, GRP)],
                             x_v.at[pl.ds(GRP * g, GRP)], x_sems[g])

    @pl.when(is_tail)
    def _stage_tail():
        pltpu.async_copy(ids_hbm.at[pl.ds(31 * CHUNK, TG0)], i0, si)
        pltpu.async_copy(ids_hbm.at[pl.ds(31 * CHUNK + TG0, TG1)], it_v, si)
        pltpu.async_copy(x_hbm.at[pl.ds(31 * CHUNK, TG0)],
                         x_v.at[pl.ds(0, TG0)], s0)
        pltpu.async_copy(x_hbm.at[pl.ds(31 * CHUNK + TG0, TG1)],
                         x_v.at[pl.ds(TG0, TG1)], s1)

    zvec = jnp.zeros((16,), jnp.float32)
    for r in range(G):
        for j in range(D // 16):
            zs_v[r, pl.ds(j * 16, 16)] = zvec
    pltpu.sync_copy(zs_v, acc_sh.at[pl.ds(slab, G)])

    def _scatter(nids, buf, xoff, nrows, xsem):
        for v in range(nids // 16):
            buf[pl.ds(16 * v, 16)] = buf[pl.ds(16 * v, 16)] + slab
        pltpu.make_async_copy(x_hbm.at[pl.ds(base + xoff, nrows)],
                              x_v.at[pl.ds(xoff, nrows)], xsem).wait()
        return pltpu.async_copy(x_v.at[pl.ds(xoff, nrows)],
                                acc_sh.at[buf], ss, add=True)

    @pl.when(~is_tail)
    def _main():
        for g in range(NG):
            pltpu.make_async_copy(ids_hbm.at[pl.ds(base + GRP * g, GRP)],
                                  idx_bufs[g], si).wait()
        ds = [_scatter(GRP, idx_bufs[g], GRP * g, GRP, x_sems[g])
              for g in range(NG)]
        for d in ds:
            d.wait()

    @pl.when(is_tail)
    def _main_tail():
        pltpu.make_async_copy(ids_hbm.at[pl.ds(31 * CHUNK, TG0)],
                              i0, si).wait()
        pltpu.make_async_copy(ids_hbm.at[pl.ds(31 * CHUNK + TG0, TG1)],
                              it_v, si).wait()
        d0 = _scatter(TG0, i0, 0, TG0, s0)
        d1 = _scatter(TG1, it_v, TG0, TG1, s1)
        d0.wait()
        d1.wait()

    plsc.subcore_barrier()

    for s in range(NS):
        pltpu.async_copy(acc_sh.at[s * G + sid], zs_v.at[s], sg)
    for s in range(NS):
        pltpu.make_async_copy(acc_sh.at[s * G + sid], zs_v.at[s], sg).wait()
    for j in range(D // 16):
        t = zs_v[0, pl.ds(16 * j, 16)]
        for s in range(1, NS):
            t = t + zs_v[s, pl.ds(16 * j, 16)]
        row_v[pl.ds(16 * j, 16)] = t
    pltpu.sync_copy(row_v, sums_out.at[cid * G + sid])


def _tc_finish(sums_ref, ids_ref, WcT_ref, bcT_ref, outT_ref):
    s = sums_ref[0:G] + sums_ref[G:2 * G]
    cnt = jnp.zeros((G, 1), jnp.float32)
    gids = jax.lax.broadcasted_iota(jnp.int32, (G, N // 10), 0)
    for b in range(10):
        ids = ids_ref[b]
        cnt = cnt + jnp.sum((gids == ids).astype(jnp.float32),
                            axis=1, keepdims=True)
    recip = 1.0 / jnp.maximum(cnt, 1.0)
    pooled = s * recip
    outT_ref[...] = jax.lax.dot_general(
        WcT_ref[...], pooled, (((1,), (1,)), ((), ())),
        preferred_element_type=jnp.float32) + bcT_ref[...]


def kernel(x, edge_index, edge_attr, batch_size, W1, b1, W2, b2, Wc, bc):
    sums = _sc_segment_sums(x, batch_size)
    ids3 = batch_size.reshape(10, 1, N // 10)
    outT = pl.pallas_call(
        _tc_finish,
        out_shape=jax.ShapeDtypeStruct((C, G), jnp.float32),
    )(sums, ids3, Wc.T, bc.reshape(C, 1))
    return outT.T


# speedup vs baseline: 1.0284x; 1.0284x over previous
import functools

import jax
import jax.numpy as jnp
from jax import lax
from jax.experimental import pallas as pl
from jax.experimental.pallas import tpu as pltpu
from jax.experimental.pallas import tpu_sc as plsc

N, D, G, C = 10000, 128, 16, 7
NW = 32
CHUNK = 320
TAIL = N - 31 * CHUNK
GRP = 64
NG = CHUNK // GRP
NGT = TAIL // GRP + 1
TG0, TG1 = 64, TAIL - 64

_MESH = plsc.VectorSubcoreMesh(core_axis_name="c", subcore_axis_name="s")


@functools.partial(
    pl.kernel,
    mesh=_MESH,
    out_type=jax.ShapeDtypeStruct((2, G, D), jnp.float32),
    scratch_types=(
        [pltpu.VMEM((CHUNK, D), jnp.float32)]
        + [pltpu.VMEM((GRP,), jnp.int32) for _ in range(NG)]
        + [pltpu.VMEM((TG1,), jnp.int32)]
        + [pltpu.VMEM((G, D), jnp.float32)]
        + [pltpu.VMEM_SHARED((G, D), jnp.float32)]
        + [pltpu.SemaphoreType.DMA for _ in range(NG)]
        + [pltpu.SemaphoreType.DMA,
           pltpu.SemaphoreType.DMA]
    ),
)
def _sc_segment_sums(x_hbm, ids_hbm, sums_out,
                     x_v, i0, i1, i2, i3, i4, it_v, zs_v, acc_sh,
                     s0, s1, s2, s3, s4, si, ss):
    idx_bufs = [i0, i1, i2, i3, i4]
    x_sems = [s0, s1, s2, s3, s4]
    cid = lax.axis_index("c")
    sid = lax.axis_index("s")
    wid = sid * 2 + cid
    base = wid * CHUNK
    is_tail = wid == NW - 1

    @pl.when(~is_tail)
    def _stage():
        for g in range(NG):
            pltpu.async_copy(ids_hbm.at[pl.ds(base + GRP * g, GRP)],
                             idx_bufs[g], si)
        for g in range(NG):
            pltpu.async_copy(x_hbm.at[pl.ds(base + GRP * g, GRP)],
                             x_v.at[pl.ds(GRP * g, GRP)], x_sems[g])

    @pl.when(is_tail)
    def _stage_tail():
        pltpu.async_copy(ids_hbm.at[pl.ds(31 * CHUNK, TG0)], i0, si)
        pltpu.async_copy(ids_hbm.at[pl.ds(31 * CHUNK + TG0, TG1)], it_v, si)
        pltpu.async_copy(x_hbm.at[pl.ds(31 * CHUNK, TG0)],
                         x_v.at[pl.ds(0, TG0)], s0)
        pltpu.async_copy(x_hbm.at[pl.ds(31 * CHUNK + TG0, TG1)],
                         x_v.at[pl.ds(TG0, TG1)], s1)

    @pl.when(sid == 0)
    def _init():
        zvec = jnp.zeros((16,), jnp.float32)
        for r in range(G):
            for j in range(D // 16):
                zs_v[r, pl.ds(j * 16, 16)] = zvec
        pltpu.sync_copy(zs_v, acc_sh)

    plsc.subcore_barrier()

    @pl.when(~is_tail)
    def _main():
        for g in range(NG):
            pltpu.make_async_copy(ids_hbm.at[pl.ds(base + GRP * g, GRP)],
                                  idx_bufs[g], si).wait()
        ds = []
        for g in range(NG):
            pltpu.make_async_copy(x_hbm.at[pl.ds(base + GRP * g, GRP)],
                                  x_v.at[pl.ds(GRP * g, GRP)],
                                  x_sems[g]).wait()
            ds.append(pltpu.async_copy(x_v.at[pl.ds(GRP * g, GRP)],
                                       acc_sh.at[idx_bufs[g]], ss,
                                       add=True))
        for d in ds:
            d.wait()

    @pl.when(is_tail)
    def _main_tail():
        pltpu.make_async_copy(ids_hbm.at[pl.ds(31 * CHUNK, TG0)],
                              i0, si).wait()
        pltpu.make_async_copy(ids_hbm.at[pl.ds(31 * CHUNK + TG0, TG1)],
                              it_v, si).wait()
        pltpu.make_async_copy(x_hbm.at[pl.ds(31 * CHUNK, TG0)],
                              x_v.at[pl.ds(0, TG0)], s0).wait()
        d0 = pltpu.async_copy(x_v.at[pl.ds(0, TG0)], acc_sh.at[i0], ss,
                              add=True)
        pltpu.make_async_copy(x_hbm.at[pl.ds(31 * CHUNK + TG0, TG1)],
                              x_v.at[pl.ds(TG0, TG1)], s1).wait()
        d1 = pltpu.async_copy(x_v.at[pl.ds(TG0, TG1)], acc_sh.at[it_v], ss,
                              add=True)
        d0.wait()
        d1.wait()

    plsc.subcore_barrier()

    @pl.when(sid == 0)
    def _flush():
        pltpu.sync_copy(acc_sh, zs_v)
        pltpu.sync_copy(zs_v, sums_out.at[cid])


def _tc_counts(ids_ref, recip_ref):
    cnt = jnp.zeros((G, 1), jnp.float32)
    gids = jax.lax.broadcasted_iota(jnp.int32, (G, N // 10), 0)
    for b in range(10):
        ids = ids_ref[b]
        cnt = cnt + jnp.sum((gids == ids).astype(jnp.float32),
                            axis=1, keepdims=True)
    recip = 1.0 / jnp.maximum(cnt, 1.0)
    recip_ref[...] = jnp.broadcast_to(recip, (G, D))


def _tc_finish(sums_ref, recip_ref, WcT_ref, bcT_ref, outT_ref):
    pooled = (sums_ref[0] + sums_ref[1]) * recip_ref[...]
    outT_ref[...] = jax.lax.dot_general(
        WcT_ref[...], pooled, (((1,), (1,)), ((), ())),
        preferred_element_type=jnp.float32) + bcT_ref[...]


def kernel(x, edge_index, edge_attr, batch_size, W1, b1, W2, b2, Wc, bc):
    sums = _sc_segment_sums(x, batch_size)
    ids3 = batch_size.reshape(10, 1, N // 10)
    recip = pl.pallas_call(
        _tc_counts,
        out_shape=jax.ShapeDtypeStruct((G, D), jnp.float32),
    )(ids3)
    outT = pl.pallas_call(
        _tc_finish,
        out_shape=jax.ShapeDtypeStruct((C, G), jnp.float32),
    )(sums, recip, Wc.T, bc.reshape(C, 1))
    return outT.T


# speedup vs baseline: 1.0284x; 1.0001x over previous
import functools

import jax
import jax.numpy as jnp
from jax import lax
from jax.experimental import pallas as pl
from jax.experimental.pallas import tpu as pltpu
from jax.experimental.pallas import tpu_sc as plsc

N, D, G, C = 10000, 128, 16, 7
NW = 32
CHUNK = 320
TAIL = N - 31 * CHUNK
GRP = 64
NG = CHUNK // GRP
TG0, TG1 = 64, TAIL - 64

_MESH = plsc.VectorSubcoreMesh(core_axis_name="c", subcore_axis_name="s")


@functools.partial(
    pl.kernel,
    mesh=_MESH,
    out_type=jax.ShapeDtypeStruct((2, G, D), jnp.float32),
    scratch_types=(
        [pltpu.VMEM((CHUNK, D), jnp.float32)]
        + [pltpu.VMEM((GRP,), jnp.int32) for _ in range(NG)]
        + [pltpu.VMEM((TG1,), jnp.int32)]
        + [pltpu.VMEM((G, D), jnp.float32)]
        + [pltpu.VMEM_SHARED((G, D), jnp.float32)]
        + [pltpu.SemaphoreType.DMA for _ in range(NG)]
        + [pltpu.SemaphoreType.DMA,
           pltpu.SemaphoreType.DMA]
    ),
)
def _sc_segment_sums(x_hbm, ids_hbm, sums_out,
                     x_v, i0, i1, i2, i3, i4, it_v, zs_v, acc_sh,
                     s0, s1, s2, s3, s4, si, ss):
    idx_bufs = [i0, i1, i2, i3, i4]
    x_sems = [s0, s1, s2, s3, s4]
    cid = lax.axis_index("c")
    sid = lax.axis_index("s")
    wid = sid * 2 + cid
    base = wid * CHUNK
    is_tail = wid == NW - 1

    @pl.when(~is_tail)
    def _stage():
        for g in range(NG):
            pltpu.async_copy(ids_hbm.at[pl.ds(base + GRP * g, GRP)],
                             idx_bufs[g], si)
        for g in range(NG):
            pltpu.async_copy(x_hbm.at[pl.ds(base + GRP * g, GRP)],
                             x_v.at[pl.ds(GRP * g, GRP)], x_sems[g])

    @pl.when(is_tail)
    def _stage_tail():
        pltpu.async_copy(ids_hbm.at[pl.ds(31 * CHUNK, TG0)], i0, si)
        pltpu.async_copy(ids_hbm.at[pl.ds(31 * CHUNK + TG0, TG1)], it_v, si)
        pltpu.async_copy(x_hbm.at[pl.ds(31 * CHUNK, TG0)],
                         x_v.at[pl.ds(0, TG0)], s0)
        pltpu.async_copy(x_hbm.at[pl.ds(31 * CHUNK + TG0, TG1)],
                         x_v.at[pl.ds(TG0, TG1)], s1)

    @pl.when(sid == 0)
    def _init():
        zvec = jnp.zeros((16,), jnp.float32)
        for r in range(G):
            for j in range(D // 16):
                zs_v[r, pl.ds(j * 16, 16)] = zvec
        pltpu.sync_copy(zs_v, acc_sh)

    plsc.subcore_barrier()

    @pl.when(~is_tail)
    def _main():
        for g in range(NG):
            pltpu.make_async_copy(ids_hbm.at[pl.ds(base + GRP * g, GRP)],
                                  idx_bufs[g], si).wait()
        ds = []
        for g in range(NG):
            pltpu.make_async_copy(x_hbm.at[pl.ds(base + GRP * g, GRP)],
                                  x_v.at[pl.ds(GRP * g, GRP)],
                                  x_sems[g]).wait()
            ds.append(pltpu.async_copy(x_v.at[pl.ds(GRP * g, GRP)],
                                       acc_sh.at[idx_bufs[g]], ss,
                                       add=True))
        for d in ds:
            d.wait()

    @pl.when(is_tail)
    def _main_tail():
        pltpu.make_async_copy(ids_hbm.at[pl.ds(31 * CHUNK, TG0)],
                              i0, si).wait()
        pltpu.make_async_copy(ids_hbm.at[pl.ds(31 * CHUNK + TG0, TG1)],
                              it_v, si).wait()
        pltpu.make_async_copy(x_hbm.at[pl.ds(31 * CHUNK, TG0)],
                              x_v.at[pl.ds(0, TG0)], s0).wait()
        d0 = pltpu.async_copy(x_v.at[pl.ds(0, TG0)], acc_sh.at[i0], ss,
                              add=True)
        pltpu.make_async_copy(x_hbm.at[pl.ds(31 * CHUNK + TG0, TG1)],
                              x_v.at[pl.ds(TG0, TG1)], s1).wait()
        d1 = pltpu.async_copy(x_v.at[pl.ds(TG0, TG1)], acc_sh.at[it_v], ss,
                              add=True)
        d0.wait()
        d1.wait()

    plsc.subcore_barrier()

    @pl.when(sid == 0)
    def _flush():
        pltpu.sync_copy(acc_sh, zs_v)
        pltpu.sync_copy(zs_v, sums_out.at[cid])


def _tc_counts(ids_ref, recip_ref):
    cnt = jnp.zeros((G, 1), jnp.float32)
    gids = jax.lax.broadcasted_iota(jnp.int32, (G, N // 10), 0)
    for b in range(10):
        ids = ids_ref[b]
        cnt = cnt + jnp.sum((gids == ids).astype(jnp.float32),
                            axis=1, keepdims=True)
    recip = 1.0 / jnp.maximum(cnt, 1.0)
    recip_ref[...] = jnp.broadcast_to(recip, (G, D))


def _tc_finish(sums_ref, recip_ref, WcT_ref, bcT_ref, outT_ref):
    pooled = (sums_ref[0] + sums_ref[1]) * recip_ref[...]
    outT_ref[...] = jax.lax.dot_general(
        WcT_ref[...], pooled, (((1,), (1,)), ((), ())),
        preferred_element_type=jnp.float32) + bcT_ref[...]


def kernel(x, edge_index, edge_attr, batch_size, W1, b1, W2, b2, Wc, bc):
    sums = _sc_segment_sums(x, batch_size)
    ids3 = batch_size.reshape(10, 1, N // 10)
    recip = pl.pallas_call(
        _tc_counts,
        out_shape=jax.ShapeDtypeStruct((G, D), jnp.float32),
    )(ids3)
    outT = pl.pallas_call(
        _tc_finish,
        out_shape=jax.ShapeDtypeStruct((C, G), jnp.float32),
    )(sums, recip, Wc.T, bc.reshape(C, 1))
    return outT.T
